# Initial kernel scaffold; baseline (speedup 1.0000x reference)
import jax
import jax.numpy as jnp
from jax.experimental import pallas as pl


def kernel(x, table):
    raise NotImplementedError("write your pallas kernel here")


# speedup vs baseline: 5.0589x; 5.0589x over previous
import functools

import jax
import jax.numpy as jnp
from jax import lax
from jax.experimental import pallas as pl
from jax.experimental.pallas import tpu as pltpu
from jax.experimental.pallas import tpu_sc as plsc

VOCAB_ROWS = 500
DIM = 64

_info = plsc.get_sparse_core_info()
NC, NS = _info.num_cores, _info.num_subcores
NW = NC * NS

CHUNK = 512


def _make_gather(total_rows: int):
    assert total_rows % (NW * CHUNK) == 0
    rows_per_w = total_rows // NW
    n_chunks = rows_per_w // CHUNK
    mesh = plsc.VectorSubcoreMesh(core_axis_name="c", subcore_axis_name="s")

    @functools.partial(
        pl.kernel,
        mesh=mesh,
        compiler_params=pltpu.CompilerParams(use_tc_tiling_on_sc=False),
        out_type=jax.ShapeDtypeStruct((total_rows, DIM), jnp.float32),
        scratch_types=[
            pltpu.VMEM((CHUNK,), jnp.int32),
            pltpu.VMEM((CHUNK, DIM), jnp.float32),
            pltpu.VMEM_SHARED((VOCAB_ROWS, DIM), jnp.float32),
            pltpu.SemaphoreType.DMA,
        ],
    )
    def gather_kernel(idx_hbm, table_hbm, out_hbm, idx_v, rows_v, table_sp, sem):
        cid = lax.axis_index("c")
        sid = lax.axis_index("s")
        wid = sid * NC + cid
        w_base = wid * rows_per_w

        @pl.when(sid == 0)
        def _():
            pltpu.sync_copy(table_hbm, table_sp)

        plsc.subcore_barrier()

        def body(i, _):
            base = w_base + i * CHUNK
            pltpu.sync_copy(idx_hbm.at[pl.ds(base, CHUNK)], idx_v)
            pltpu.async_copy(table_sp.at[idx_v], rows_v, sem).wait()
            pltpu.sync_copy(rows_v, out_hbm.at[pl.ds(base, CHUNK)])
            return ()

        lax.fori_loop(0, n_chunks, body, (), unroll=False)

    return gather_kernel


def kernel(x, table):
    b, s = x.shape
    idx = x.reshape(b * s).astype(jnp.int32)
    eff_table = table.at[0].set(0.0)
    out = _make_gather(b * s)(idx, eff_table)
    return out.reshape(b, s, DIM)


# speedup vs baseline: 5.6870x; 1.1242x over previous
import functools

import jax
import jax.numpy as jnp
from jax import lax
from jax.experimental import pallas as pl
from jax.experimental.pallas import tpu as pltpu
from jax.experimental.pallas import tpu_sc as plsc

VOCAB_ROWS = 500
DIM = 64

_info = plsc.get_sparse_core_info()
NC, NS = _info.num_cores, _info.num_subcores
NW = NC * NS

CHUNK = 512


def _make_gather(total_rows: int):
    assert total_rows % (NW * 2 * CHUNK) == 0
    rows_per_w = total_rows // NW
    n_pairs = rows_per_w // (2 * CHUNK)
    mesh = plsc.VectorSubcoreMesh(core_axis_name="c", subcore_axis_name="s")

    @functools.partial(
        pl.kernel,
        mesh=mesh,
        compiler_params=pltpu.CompilerParams(use_tc_tiling_on_sc=False),
        out_type=jax.ShapeDtypeStruct((total_rows, DIM), jnp.float32),
        scratch_types=[
            pltpu.VMEM_SHARED((VOCAB_ROWS, DIM), jnp.float32),
            pltpu.VMEM((2 * CHUNK,), jnp.int32),
            pltpu.VMEM((2, CHUNK, DIM), jnp.float32),
            pltpu.SemaphoreType.DMA,
            pltpu.SemaphoreType.DMA,
            pltpu.SemaphoreType.DMA,
            pltpu.SemaphoreType.DMA,
        ],
    )
    def gather_kernel(idx_hbm, table_hbm, out_hbm, table_sp, idx_v, rows_v,
                      sem_g0, sem_g1, sem_o0, sem_o1):
        cid = lax.axis_index("c")
        sid = lax.axis_index("s")
        wid = sid * NC + cid
        w_base = wid * rows_per_w

        @pl.when(sid == 0)
        def _():
            pltpu.sync_copy(table_hbm, table_sp)

        plsc.subcore_barrier()

        def load_idx(pair):
            pltpu.sync_copy(
                idx_hbm.at[pl.ds(w_base + pair * 2 * CHUNK, 2 * CHUNK)], idx_v)

        def start_gather(half, sem):
            return pltpu.async_copy(
                table_sp.at[idx_v.at[pl.ds(half * CHUNK, CHUNK)]],
                rows_v.at[half], sem)

        def start_store(pair, half, sem):
            return pltpu.async_copy(
                rows_v.at[half],
                out_hbm.at[pl.ds(w_base + (pair * 2 + half) * CHUNK, CHUNK)],
                sem)

        def wait_store(half, sem):
            pltpu.make_async_copy(
                rows_v.at[half], out_hbm.at[pl.ds(w_base, CHUNK)], sem).wait()

        load_idx(0)
        start_gather(0, sem_g0).wait()
        h1 = start_gather(1, sem_g1)
        start_store(0, 0, sem_o0)
        h1.wait()
        start_store(0, 1, sem_o1)

        def body(g, _):
            load_idx(g)
            wait_store(0, sem_o0)
            h0 = start_gather(0, sem_g0)
            h0.wait()
            wait_store(1, sem_o1)
            h1 = start_gather(1, sem_g1)
            start_store(g, 0, sem_o0)
            h1.wait()
            start_store(g, 1, sem_o1)
            return ()

        lax.fori_loop(1, n_pairs, body, (), unroll=False)

        wait_store(0, sem_o0)
        wait_store(1, sem_o1)

    return gather_kernel


def kernel(x, table):
    b, s = x.shape
    idx = x.reshape(b * s).astype(jnp.int32)
    eff_table = table.at[0].set(0.0)
    out = _make_gather(b * s)(idx, eff_table)
    return out.reshape(b, s, DIM)
